# baseline (speedup 1.0000x reference)
import jax
import jax.numpy as jnp
from jax.experimental import pallas as pl
from jax.experimental.pallas import tpu as pltpu

EPS = 1e-5

D = 128
BM = 400
BM2 = 1000


def _pass1_body(adj_ref, xf_ref, s01_ref, w01_ref, w1b_ref, sc_ref, sh_ref,
                out_ref, y1a_ref, s1b_ref, q_ref, cs_ref, sup_scr):
    i = pl.program_id(0)

    @pl.when(i == 0)
    def _():
        sup_scr[...] = jnp.dot(xf_ref[...], w01_ref[...],
                               preferred_element_type=jnp.float32
                               ).astype(jnp.bfloat16)

    adj = adj_ref[...]
    x_blk = xf_ref[pl.ds(i * BM, BM), :]
    acc = jnp.dot(adj.astype(jnp.bfloat16), sup_scr[...],
                  preferred_element_type=jnp.float32)
    acc = acc + jnp.dot(x_blk, s01_ref[...],
                        preferred_element_type=jnp.float32)
    y = jnp.maximum(acc * sc_ref[...] + sh_ref[...], 0.0)
    out_ref[...] = jnp.concatenate([x_blk, y[:, :D]], axis=1)
    y1a_ref[...] = y[:, D:].astype(jnp.bfloat16)
    s1b = jnp.dot(y[:, D:], w1b_ref[...],
                  preferred_element_type=jnp.float32)
    s1b_ref[...] = s1b.astype(jnp.bfloat16)
    col = jnp.sum(s1b, axis=0, keepdims=True)
    cs_ref[...] = jnp.where(i == 0, col, cs_ref[...] + col)
    q_ref[...] = (jnp.round(adj * 255.0) - 128.0).astype(jnp.int8)


def _pass2_body(q_ref, s1b_ref, y1a_ref, s1_ref, cs_ref, sc_ref, sh_ref,
                obuf_ref, y1b_ref):
    del obuf_ref
    acc = jnp.dot(q_ref[...].astype(jnp.bfloat16), s1b_ref[...],
                  preferred_element_type=jnp.float32) * (1.0 / 255.0)
    acc = acc + cs_ref[...]
    acc = acc + jnp.dot(y1a_ref[...], s1_ref[...],
                        preferred_element_type=jnp.float32)
    y1b_ref[...] = jnp.maximum(acc * sc_ref[...] + sh_ref[...], 0.0)


def kernel(input, adj, W_l0, S_l0, b_l0, g_l0, be_l0, m_l0, v_l0,
           W_l1a, S_l1a, b_l1a, g_l1a, be_l1a, m_l1a, v_l1a,
           W_l1b, S_l1b, b_l1b, g_l1b, be_l1b, m_l1b, v_l1b):
    x = input
    n = x.shape[0]

    def fold(b, g, be, m, v):
        sc = g * jax.lax.rsqrt(v + EPS)
        return sc, be + (b - m) * sc

    sc0, sh0 = fold(b_l0, g_l0, be_l0, m_l0, v_l0)
    sc1a, sh1a = fold(b_l1a, g_l1a, be_l1a, m_l1a, v_l1a)
    sc1b, sh1b = fold(b_l1b, g_l1b, be_l1b, m_l1b, v_l1b)

    W01 = jnp.concatenate([W_l0, W_l1a], axis=1)
    S01 = jnp.concatenate([S_l0, S_l1a], axis=1)
    sc01 = jnp.concatenate([sc0, sc1a])[None, :]
    sh01 = jnp.concatenate([sh0, sh1a])[None, :]

    grid = (n // BM,)
    out01, y1a, s1b, adj_q, cs_raw = pl.pallas_call(
        _pass1_body,
        grid=grid,
        in_specs=[
            pl.BlockSpec((BM, n), lambda i: (i, 0)),
            pl.BlockSpec((n, D), lambda i: (0, 0)),
            pl.BlockSpec((D, 2 * D), lambda i: (0, 0)),
            pl.BlockSpec((D, 2 * D), lambda i: (0, 0)),
            pl.BlockSpec((D, D), lambda i: (0, 0)),
            pl.BlockSpec((1, 2 * D), lambda i: (0, 0)),
            pl.BlockSpec((1, 2 * D), lambda i: (0, 0)),
        ],
        out_specs=[
            pl.BlockSpec((BM, 2 * D), lambda i: (i, 0)),
            pl.BlockSpec((BM, D), lambda i: (i, 0)),
            pl.BlockSpec((BM, D), lambda i: (i, 0)),
            pl.BlockSpec((BM, n), lambda i: (i, 0)),
            pl.BlockSpec((1, D), lambda i: (0, 0)),
        ],
        out_shape=[
            jax.ShapeDtypeStruct((n, 3 * D), jnp.float32),
            jax.ShapeDtypeStruct((n, D), jnp.bfloat16),
            jax.ShapeDtypeStruct((n, D), jnp.bfloat16),
            jax.ShapeDtypeStruct((n, n), jnp.int8),
            jax.ShapeDtypeStruct((1, D), jnp.float32),
        ],
        scratch_shapes=[pltpu.VMEM((n, 2 * D), jnp.bfloat16)],
    )(adj, x, S01, W01, W_l1b, sc01, sh01)

    out = pl.pallas_call(
        _pass2_body,
        grid=(n // BM2,),
        in_specs=[
            pl.BlockSpec((BM2, n), lambda i: (i, 0)),
            pl.BlockSpec((n, D), lambda i: (0, 0)),
            pl.BlockSpec((BM2, D), lambda i: (i, 0)),
            pl.BlockSpec((D, D), lambda i: (0, 0)),
            pl.BlockSpec((1, D), lambda i: (0, 0)),
            pl.BlockSpec((1, D), lambda i: (0, 0)),
            pl.BlockSpec((1, D), lambda i: (0, 0)),
            pl.BlockSpec((8, D), lambda i: (0, 0)),
        ],
        out_specs=pl.BlockSpec((BM2, D), lambda i: (i, 2)),
        out_shape=jax.ShapeDtypeStruct((n, 3 * D), jnp.float32),
        input_output_aliases={7: 0},
    )(adj_q, s1b, y1a, S_l1b.astype(jnp.bfloat16),
      cs_raw * (128.0 / 255.0), sc1b[None, :], sh1b[None, :], out01)

    return out


# speedup vs baseline: 1.0043x; 1.0043x over previous
import jax
import jax.numpy as jnp
from jax.experimental import pallas as pl
from jax.experimental.pallas import tpu as pltpu

EPS = 1e-5

D = 128
BM = 400
BM2 = 1000


def _pass1_body(adj_ref, xf_ref, s01_ref, w01_ref, w1b_ref, sc_ref, sh_ref,
                out_ref, s1b_ref, q_ref, cs_ref, sup_scr):
    i = pl.program_id(0)

    @pl.when(i == 0)
    def _():
        sup_scr[...] = jnp.dot(xf_ref[...], w01_ref[...],
                               preferred_element_type=jnp.float32
                               ).astype(jnp.bfloat16)

    adj = adj_ref[...]
    x_blk = xf_ref[pl.ds(i * BM, BM), :]
    acc = jnp.dot(adj.astype(jnp.bfloat16), sup_scr[...],
                  preferred_element_type=jnp.float32)
    acc = acc + jnp.dot(x_blk, s01_ref[...],
                        preferred_element_type=jnp.float32)
    y = jnp.maximum(acc * sc_ref[...] + sh_ref[...], 0.0)
    out_ref[...] = jnp.concatenate([x_blk, y], axis=1)
    s1b = jnp.dot(y[:, D:], w1b_ref[...],
                  preferred_element_type=jnp.float32)
    s1b_ref[...] = s1b.astype(jnp.bfloat16)
    col = jnp.sum(s1b, axis=0, keepdims=True)
    cs_ref[...] = jnp.where(i == 0, col, cs_ref[...] + col)
    q_ref[...] = (jnp.round(adj * 255.0) - 128.0).astype(jnp.int8)


def _pass2_body(q_ref, s1b_ref, y1a_ref, s1_ref, cs_ref, sc_ref, sh_ref,
                y1b_ref):
    acc = jnp.dot(q_ref[...].astype(jnp.bfloat16), s1b_ref[...],
                  preferred_element_type=jnp.float32) * (1.0 / 255.0)
    acc = acc + cs_ref[...]
    acc = acc + jnp.dot(y1a_ref[...], s1_ref[...],
                        preferred_element_type=jnp.float32)
    y1b_ref[...] = jnp.maximum(acc * sc_ref[...] + sh_ref[...], 0.0)


def kernel(input, adj, W_l0, S_l0, b_l0, g_l0, be_l0, m_l0, v_l0,
           W_l1a, S_l1a, b_l1a, g_l1a, be_l1a, m_l1a, v_l1a,
           W_l1b, S_l1b, b_l1b, g_l1b, be_l1b, m_l1b, v_l1b):
    x = input
    n = x.shape[0]

    def fold(b, g, be, m, v):
        sc = g * jax.lax.rsqrt(v + EPS)
        return sc, be + (b - m) * sc

    sc0, sh0 = fold(b_l0, g_l0, be_l0, m_l0, v_l0)
    sc1a, sh1a = fold(b_l1a, g_l1a, be_l1a, m_l1a, v_l1a)
    sc1b, sh1b = fold(b_l1b, g_l1b, be_l1b, m_l1b, v_l1b)

    W01 = jnp.concatenate([W_l0, W_l1a], axis=1)
    S01 = jnp.concatenate([S_l0, S_l1a], axis=1)
    sc01 = jnp.concatenate([sc0, sc1a])[None, :]
    sh01 = jnp.concatenate([sh0, sh1a])[None, :]

    grid = (n // BM,)
    out01, s1b, adj_q, cs_raw = pl.pallas_call(
        _pass1_body,
        grid=grid,
        in_specs=[
            pl.BlockSpec((BM, n), lambda i: (i, 0)),
            pl.BlockSpec((n, D), lambda i: (0, 0)),
            pl.BlockSpec((D, 2 * D), lambda i: (0, 0)),
            pl.BlockSpec((D, 2 * D), lambda i: (0, 0)),
            pl.BlockSpec((D, D), lambda i: (0, 0)),
            pl.BlockSpec((1, 2 * D), lambda i: (0, 0)),
            pl.BlockSpec((1, 2 * D), lambda i: (0, 0)),
        ],
        out_specs=[
            pl.BlockSpec((BM, 3 * D), lambda i: (i, 0)),
            pl.BlockSpec((BM, D), lambda i: (i, 0)),
            pl.BlockSpec((BM, n), lambda i: (i, 0)),
            pl.BlockSpec((1, D), lambda i: (0, 0)),
        ],
        out_shape=[
            jax.ShapeDtypeStruct((n, 3 * D), jnp.float32),
            jax.ShapeDtypeStruct((n, D), jnp.bfloat16),
            jax.ShapeDtypeStruct((n, n), jnp.int8),
            jax.ShapeDtypeStruct((1, D), jnp.float32),
        ],
        scratch_shapes=[pltpu.VMEM((n, 2 * D), jnp.bfloat16)],
    )(adj, x, S01, W01, W_l1b, sc01, sh01)

    out = pl.pallas_call(
        _pass2_body,
        grid=(n // BM2,),
        in_specs=[
            pl.BlockSpec((BM2, n), lambda i: (i, 0)),
            pl.BlockSpec((n, D), lambda i: (0, 0)),
            pl.BlockSpec((BM2, D), lambda i: (i, 2)),
            pl.BlockSpec((D, D), lambda i: (0, 0)),
            pl.BlockSpec((1, D), lambda i: (0, 0)),
            pl.BlockSpec((1, D), lambda i: (0, 0)),
            pl.BlockSpec((1, D), lambda i: (0, 0)),
        ],
        out_specs=pl.BlockSpec((BM2, D), lambda i: (i, 2)),
        out_shape=jax.ShapeDtypeStruct((n, 3 * D), jnp.float32),
        input_output_aliases={2: 0},
    )(adj_q, s1b, out01, S_l1b, cs_raw * (128.0 / 255.0),
      sc1b[None, :], sh1b[None, :])

    return out


# speedup vs baseline: 1.0343x; 1.0299x over previous
import jax
import jax.numpy as jnp
from jax.experimental import pallas as pl
from jax.experimental.pallas import tpu as pltpu

EPS = 1e-5

D = 128
BM = 400
BM2 = 1000


def _pass1_body(adj_ref, xf_ref, s01_ref, w01_ref, w1b_ref, sc_ref, sh_ref,
                out_ref, s1b_ref, q_ref, cs_ref, sup_scr):
    i = pl.program_id(0)

    @pl.when(i == 0)
    def _():
        sup_scr[...] = jnp.dot(xf_ref[...], w01_ref[...],
                               preferred_element_type=jnp.float32
                               ).astype(jnp.bfloat16)

    adj = adj_ref[...]
    x_blk = xf_ref[pl.ds(i * BM, BM), :]
    acc = jnp.dot(adj.astype(jnp.bfloat16), sup_scr[...],
                  preferred_element_type=jnp.float32)
    acc = acc + jnp.dot(x_blk, s01_ref[...],
                        preferred_element_type=jnp.float32)
    y = jnp.maximum(acc * sc_ref[...] + sh_ref[...], 0.0)
    out_ref[...] = jnp.concatenate([x_blk, y], axis=1)
    s1b = jnp.dot(y[:, D:], w1b_ref[...],
                  preferred_element_type=jnp.float32)
    s1b_ref[...] = s1b.astype(jnp.bfloat16)
    col = jnp.sum(s1b, axis=0, keepdims=True)
    cs_ref[...] = jnp.where(i == 0, col, cs_ref[...] + col)
    q_ref[...] = (adj * 255.0 - 128.0).astype(jnp.int8)


def _pass2_body(q_ref, s1b_ref, y1a_ref, s1_ref, cs_ref, sc_ref, sh_ref,
                y1b_ref):
    acc = jnp.dot(q_ref[...].astype(jnp.bfloat16), s1b_ref[...],
                  preferred_element_type=jnp.float32) * (1.0 / 255.0)
    acc = acc + cs_ref[...]
    acc = acc + jnp.dot(y1a_ref[...], s1_ref[...],
                        preferred_element_type=jnp.float32)
    y1b_ref[...] = jnp.maximum(acc * sc_ref[...] + sh_ref[...], 0.0)


def kernel(input, adj, W_l0, S_l0, b_l0, g_l0, be_l0, m_l0, v_l0,
           W_l1a, S_l1a, b_l1a, g_l1a, be_l1a, m_l1a, v_l1a,
           W_l1b, S_l1b, b_l1b, g_l1b, be_l1b, m_l1b, v_l1b):
    x = input
    n = x.shape[0]

    def fold(b, g, be, m, v):
        sc = g * jax.lax.rsqrt(v + EPS)
        return sc, be + (b - m) * sc

    sc0, sh0 = fold(b_l0, g_l0, be_l0, m_l0, v_l0)
    sc1a, sh1a = fold(b_l1a, g_l1a, be_l1a, m_l1a, v_l1a)
    sc1b, sh1b = fold(b_l1b, g_l1b, be_l1b, m_l1b, v_l1b)

    W01 = jnp.concatenate([W_l0, W_l1a], axis=1)
    S01 = jnp.concatenate([S_l0, S_l1a], axis=1)
    sc01 = jnp.concatenate([sc0, sc1a])[None, :]
    sh01 = jnp.concatenate([sh0, sh1a])[None, :]

    grid = (n // BM,)
    out01, s1b, adj_q, cs_raw = pl.pallas_call(
        _pass1_body,
        grid=grid,
        in_specs=[
            pl.BlockSpec((BM, n), lambda i: (i, 0)),
            pl.BlockSpec((n, D), lambda i: (0, 0)),
            pl.BlockSpec((D, 2 * D), lambda i: (0, 0)),
            pl.BlockSpec((D, 2 * D), lambda i: (0, 0)),
            pl.BlockSpec((D, D), lambda i: (0, 0)),
            pl.BlockSpec((1, 2 * D), lambda i: (0, 0)),
            pl.BlockSpec((1, 2 * D), lambda i: (0, 0)),
        ],
        out_specs=[
            pl.BlockSpec((BM, 3 * D), lambda i: (i, 0)),
            pl.BlockSpec((BM, D), lambda i: (i, 0)),
            pl.BlockSpec((BM, n), lambda i: (i, 0)),
            pl.BlockSpec((1, D), lambda i: (0, 0)),
        ],
        out_shape=[
            jax.ShapeDtypeStruct((n, 3 * D), jnp.float32),
            jax.ShapeDtypeStruct((n, D), jnp.bfloat16),
            jax.ShapeDtypeStruct((n, n), jnp.int8),
            jax.ShapeDtypeStruct((1, D), jnp.float32),
        ],
        scratch_shapes=[pltpu.VMEM((n, 2 * D), jnp.bfloat16)],
    )(adj, x, S01, W01, W_l1b, sc01, sh01)

    out = pl.pallas_call(
        _pass2_body,
        grid=(n // BM2,),
        in_specs=[
            pl.BlockSpec((BM2, n), lambda i: (i, 0)),
            pl.BlockSpec((n, D), lambda i: (0, 0)),
            pl.BlockSpec((BM2, D), lambda i: (i, 2)),
            pl.BlockSpec((D, D), lambda i: (0, 0)),
            pl.BlockSpec((1, D), lambda i: (0, 0)),
            pl.BlockSpec((1, D), lambda i: (0, 0)),
            pl.BlockSpec((1, D), lambda i: (0, 0)),
        ],
        out_specs=pl.BlockSpec((BM2, D), lambda i: (i, 2)),
        out_shape=jax.ShapeDtypeStruct((n, 3 * D), jnp.float32),
        input_output_aliases={2: 0},
    )(adj_q, s1b, out01, S_l1b, cs_raw * (128.0 / 255.0),
      sc1b[None, :], sh1b[None, :])

    return out


# speedup vs baseline: 1.0490x; 1.0141x over previous
import jax
import jax.numpy as jnp
from jax.experimental import pallas as pl
from jax.experimental.pallas import tpu as pltpu

EPS = 1e-5

D = 128
BM = 400
BM2 = 1000


def _pass1_body(adj_ref, xf_ref, s01_ref, w01_ref, w1b_ref, sc_ref, sh_ref,
                out_ref, s1b_ref, q_ref, sup_scr):
    i = pl.program_id(0)

    @pl.when(i == 0)
    def _():
        sup_scr[...] = jnp.dot(xf_ref[...], w01_ref[...],
                               preferred_element_type=jnp.float32
                               ).astype(jnp.bfloat16)

    adj = adj_ref[...]
    x_blk = xf_ref[pl.ds(i * BM, BM), :]
    acc = jnp.dot(adj.astype(jnp.bfloat16), sup_scr[...],
                  preferred_element_type=jnp.float32)
    acc = acc + jnp.dot(x_blk, s01_ref[...],
                        preferred_element_type=jnp.float32)
    y = jnp.maximum(acc * sc_ref[...] + sh_ref[...], 0.0)
    out_ref[...] = jnp.concatenate([x_blk, y], axis=1)
    s1b = jnp.dot(y[:, D:], w1b_ref[...],
                  preferred_element_type=jnp.float32)
    s1b_ref[...] = s1b.astype(jnp.bfloat16)
    q_ref[...] = (adj * 255.0).astype(jnp.uint8)


def _pass2_body(q_ref, s1b_ref, y1a_ref, s1_ref, sc_ref, sh_ref,
                y1b_ref):
    acc = jnp.dot(q_ref[...].astype(jnp.bfloat16), s1b_ref[...],
                  preferred_element_type=jnp.float32) * (1.0 / 255.0)
    acc = acc + jnp.dot(y1a_ref[...], s1_ref[...],
                        preferred_element_type=jnp.float32)
    y1b_ref[...] = jnp.maximum(acc * sc_ref[...] + sh_ref[...], 0.0)


def kernel(input, adj, W_l0, S_l0, b_l0, g_l0, be_l0, m_l0, v_l0,
           W_l1a, S_l1a, b_l1a, g_l1a, be_l1a, m_l1a, v_l1a,
           W_l1b, S_l1b, b_l1b, g_l1b, be_l1b, m_l1b, v_l1b):
    x = input
    n = x.shape[0]

    def fold(b, g, be, m, v):
        sc = g * jax.lax.rsqrt(v + EPS)
        return sc, be + (b - m) * sc

    sc0, sh0 = fold(b_l0, g_l0, be_l0, m_l0, v_l0)
    sc1a, sh1a = fold(b_l1a, g_l1a, be_l1a, m_l1a, v_l1a)
    sc1b, sh1b = fold(b_l1b, g_l1b, be_l1b, m_l1b, v_l1b)

    W01 = jnp.concatenate([W_l0, W_l1a], axis=1)
    S01 = jnp.concatenate([S_l0, S_l1a], axis=1)
    sc01 = jnp.concatenate([sc0, sc1a])[None, :]
    sh01 = jnp.concatenate([sh0, sh1a])[None, :]

    grid = (n // BM,)
    out01, s1b, adj_q = pl.pallas_call(
        _pass1_body,
        grid=grid,
        in_specs=[
            pl.BlockSpec((BM, n), lambda i: (i, 0)),
            pl.BlockSpec((n, D), lambda i: (0, 0)),
            pl.BlockSpec((D, 2 * D), lambda i: (0, 0)),
            pl.BlockSpec((D, 2 * D), lambda i: (0, 0)),
            pl.BlockSpec((D, D), lambda i: (0, 0)),
            pl.BlockSpec((1, 2 * D), lambda i: (0, 0)),
            pl.BlockSpec((1, 2 * D), lambda i: (0, 0)),
        ],
        out_specs=[
            pl.BlockSpec((BM, 3 * D), lambda i: (i, 0)),
            pl.BlockSpec((BM, D), lambda i: (i, 0)),
            pl.BlockSpec((BM, n), lambda i: (i, 0)),
        ],
        out_shape=[
            jax.ShapeDtypeStruct((n, 3 * D), jnp.float32),
            jax.ShapeDtypeStruct((n, D), jnp.bfloat16),
            jax.ShapeDtypeStruct((n, n), jnp.uint8),
        ],
        scratch_shapes=[pltpu.VMEM((n, 2 * D), jnp.bfloat16)],
    )(adj, x, S01, W01, W_l1b, sc01, sh01)

    out = pl.pallas_call(
        _pass2_body,
        grid=(n // BM2,),
        in_specs=[
            pl.BlockSpec((BM2, n), lambda i: (i, 0)),
            pl.BlockSpec((n, D), lambda i: (0, 0)),
            pl.BlockSpec((BM2, D), lambda i: (i, 2)),
            pl.BlockSpec((D, D), lambda i: (0, 0)),
            pl.BlockSpec((1, D), lambda i: (0, 0)),
            pl.BlockSpec((1, D), lambda i: (0, 0)),
        ],
        out_specs=pl.BlockSpec((BM2, D), lambda i: (i, 2)),
        out_shape=jax.ShapeDtypeStruct((n, 3 * D), jnp.float32),
        input_output_aliases={2: 0},
    )(adj_q, s1b, out01, S_l1b, sc1b[None, :], sh1b[None, :])

    return out
